# baseline (speedup 1.0000x reference)
import functools

import jax
import jax.numpy as jnp
from jax import lax
from jax.experimental import pallas as pl
from jax.experimental.pallas import tpu as pltpu
from jax.experimental.pallas import tpu_sc as plsc

_D = 64
_DP = 128
_NW = 32
_CHUNK = 200
_NBUF = 4
_TBLK = 32768

_mesh = plsc.VectorSubcoreMesh(core_axis_name="c", subcore_axis_name="s")


@functools.lru_cache(maxsize=None)
def _build_tr(V):
    n_blk = (V + _TBLK - 1) // _TBLK

    def tk(t_ref, o_ref):
        o_ref[:, :_D] = t_ref[...].T

    return pl.pallas_call(
        tk,
        grid=(n_blk,),
        in_specs=[pl.BlockSpec((_D, _TBLK), lambda i: (0, i))],
        out_specs=pl.BlockSpec((_TBLK, _DP), lambda i: (i, 0)),
        out_shape=jax.ShapeDtypeStruct((V, _DP), jnp.float32),
    )


@functools.lru_cache(maxsize=None)
def _build(B):
    b_per_w = B // _NW
    n_chunks = b_per_w // _CHUNK

    @functools.partial(
        pl.kernel,
        mesh=_mesh,
        compiler_params=pltpu.CompilerParams(use_tc_tiling_on_sc=True),
        out_type=jax.ShapeDtypeStruct((B, _DP), jnp.float32),
        scratch_types=[
            pltpu.VMEM((b_per_w,), jnp.int32),
            pltpu.VMEM((_NBUF, _CHUNK, _DP), jnp.float32),
            pltpu.SemaphoreType.DMA,
            pltpu.SemaphoreType.DMA,
            pltpu.SemaphoreType.DMA,
            pltpu.SemaphoreType.DMA,
            pltpu.SemaphoreType.DMA,
            pltpu.SemaphoreType.DMA,
            pltpu.SemaphoreType.DMA,
            pltpu.SemaphoreType.DMA,
        ],
    )
    def emb(idx_hbm, table_hbm, out_hbm, idx_v, rows_v,
            g0, g1, g2, g3, o0, o1, o2, o3):
        gsem = (g0, g1, g2, g3)
        osem = (o0, o1, o2, o3)
        wid = lax.axis_index("s") * 2 + lax.axis_index("c")
        base = wid * b_per_w
        pltpu.sync_copy(idx_hbm.at[pl.ds(base, b_per_w)], idx_v)

        gathers = [None] * _NBUF
        outs = [None] * _NBUF
        for i in range(n_chunks):
            b = i % _NBUF
            if outs[b] is not None:
                outs[b].wait()
            gathers[b] = pltpu.async_copy(
                table_hbm.at[idx_v.at[pl.ds(i * _CHUNK, _CHUNK)]],
                rows_v.at[b], gsem[b])
            if i > 0:
                pb = (i - 1) % _NBUF
                gathers[pb].wait()
                outs[pb] = pltpu.async_copy(
                    rows_v.at[pb],
                    out_hbm.at[pl.ds(base + (i - 1) * _CHUNK, _CHUNK)],
                    osem[pb])
        last = n_chunks - 1
        lb = last % _NBUF
        gathers[lb].wait()
        outs[lb] = pltpu.async_copy(
            rows_v.at[lb],
            out_hbm.at[pl.ds(base + last * _CHUNK, _CHUNK)],
            osem[lb])
        for b in range(_NBUF):
            if outs[b] is not None:
                outs[b].wait()

    return emb


def kernel(input_ids, table):
    bt, h = input_ids.shape
    flat = input_ids.T.reshape(bt * h).astype(jnp.int32)
    tpad = _build_tr(table.shape[0])(table.T)
    out = _build(bt * h)(flat, tpad)
    return out[:, :_D].reshape(h, bt, _D).transpose(1, 0, 2)
